# baseline (speedup 1.0000x reference)
import functools

import jax
import jax.numpy as jnp
from jax import lax
from jax.experimental import pallas as pl
from jax.experimental.pallas import tpu as pltpu
from jax.experimental.pallas import tpu_sc as plsc

TOP_K = 8

_SORT8 = ((0, 1), (2, 3), (4, 5), (6, 7),
          (0, 2), (1, 3), (4, 6), (5, 7),
          (1, 2), (5, 6),
          (0, 4), (1, 5), (2, 6), (3, 7),
          (2, 4), (3, 5),
          (1, 2), (3, 4), (5, 6))
_MERGE8 = ((0, 4), (1, 5), (2, 6), (3, 7),
           (0, 2), (1, 3), (4, 6), (5, 7),
           (0, 1), (2, 3), (4, 5), (6, 7))


def _ce(vals, idxs, i, j):
    m = vals[j] > vals[i]
    vi, vj = vals[i], vals[j]
    ii, ij = idxs[i], idxs[j]
    vals[i] = jnp.where(m, vj, vi)
    vals[j] = jnp.where(m, vi, vj)
    idxs[i] = jnp.where(m, ij, ii)
    idxs[j] = jnp.where(m, ii, ij)

_NC = 2
_NS = 16
_LANES = 16
_NW = _NC * _NS

_BN = 512


def _router_logits(hidden_states, w, row_start, rows):
    t, d = hidden_states.shape
    e = w.shape[0]
    blk0 = row_start // _BN

    def body(h_ref, w_ref, o_ref):
        o_ref[...] = lax.dot_general(
            h_ref[...], w_ref[...],
            (((1,), (1,)), ((), ())),
            preferred_element_type=jnp.float32,
        )

    return pl.pallas_call(
        body,
        grid=(rows // _BN,),
        compiler_params=pltpu.CompilerParams(
            dimension_semantics=("parallel",)),
        in_specs=[
            pl.BlockSpec((_BN, d), lambda i: (blk0 + i, 0)),
            pl.BlockSpec((e, d), lambda i: (0, 0)),
        ],
        out_specs=pl.BlockSpec((_BN, e), lambda i: (i, 0)),
        out_shape=jax.ShapeDtypeStruct((rows, e), jnp.float32),
    )(hidden_states, w)


def _topk_softmax_sc(logits):
    t, e = logits.shape
    rpt = t // _NW
    groups = rpt // _LANES

    mesh = plsc.VectorSubcoreMesh(
        core_axis_name="c", subcore_axis_name="s",
        num_cores=_NC, num_subcores=_NS,
    )

    @functools.partial(
        pl.kernel,
        out_type=(
            jax.ShapeDtypeStruct((t * TOP_K,), jnp.int32),
            jax.ShapeDtypeStruct((t * TOP_K,), jnp.float32),
        ),
        mesh=mesh,
        compiler_params=pltpu.CompilerParams(needs_layout_passes=False),
        scratch_types=[
            pltpu.VMEM((rpt * e,), jnp.float32),
            pltpu.VMEM((rpt * TOP_K,), jnp.int32),
            pltpu.VMEM((rpt * TOP_K,), jnp.float32),
        ],
    )
    def run(logits_hbm, idx_hbm, gate_hbm, lg_v, idx_v, gate_v):
        wid = lax.axis_index("s") * _NC + lax.axis_index("c")
        base = wid * rpt
        pltpu.sync_copy(logits_hbm.at[pl.ds(base * e, rpt * e)], lg_v)

        def group(g, carry):
            rows = g * _LANES + lax.iota(jnp.int32, _LANES)
            rows_e = rows * e
            rows_k = rows * TOP_K
            tv = ti = None
            for b in range(e // TOP_K):
                bv = [plsc.load_gather(lg_v, [rows_e + (b * TOP_K + k)])
                      for k in range(TOP_K)]
                bi = [jnp.full((_LANES,), b * TOP_K + k, jnp.int32)
                      for k in range(TOP_K)]
                for (i, j) in _SORT8:
                    _ce(bv, bi, i, j)
                if tv is None:
                    tv, ti = bv, bi
                else:
                    cv, ci = [], []
                    for j in range(TOP_K):
                        m = tv[j] > bv[TOP_K - 1 - j]
                        cv.append(jnp.where(m, tv[j], bv[TOP_K - 1 - j]))
                        ci.append(jnp.where(m, ti[j], bi[TOP_K - 1 - j]))
                    for (i, j) in _MERGE8:
                        _ce(cv, ci, i, j)
                    tv, ti = cv, ci
            mx = tv[0]
            ex = [jnp.exp(tj - mx) for tj in tv]
            s = ex[0]
            for j in range(1, TOP_K):
                s = s + ex[j]
            inv = 1.0 / s
            for j in range(TOP_K):
                plsc.store_scatter(idx_v, [rows_k + j], ti[j])
                plsc.store_scatter(gate_v, [rows_k + j], ex[j] * inv)
            return carry

        lax.fori_loop(0, groups, group, 0)
        pltpu.sync_copy(idx_v, idx_hbm.at[pl.ds(base * TOP_K, rpt * TOP_K)])
        pltpu.sync_copy(gate_v, gate_hbm.at[pl.ds(base * TOP_K, rpt * TOP_K)])

    idx_flat, gate_flat = run(logits.reshape(t * e))
    return idx_flat.reshape(t, TOP_K), gate_flat.reshape(t, TOP_K)


def _fused_router_topk_tc(hidden_states, w, row_start, rows):
    t, d = hidden_states.shape
    e = w.shape[0]
    blk0 = row_start // _BN

    def body(h_ref, w_ref, idx_ref, gate_ref):
        step = pl.program_id(0)
        lg = lax.dot_general(
            h_ref[...], w_ref[...],
            (((1,), (1,)), ((), ())),
            preferred_element_type=jnp.float32,
        )
        lane = lax.broadcasted_iota(jnp.int32, lg.shape, 1)
        vals, idxs = [], []
        cur = lg
        for _ in range(TOP_K):
            m = jnp.max(cur, axis=1, keepdims=True)
            cand = jnp.where(cur == m, lane, jnp.int32(e))
            am = jnp.min(cand, axis=1, keepdims=True)
            vals.append(m)
            idxs.append(am)
            cur = jnp.where(lane == am, -jnp.inf, cur)
        mx = vals[0]
        exs = [jnp.exp(v - mx) for v in vals]
        s = exs[0]
        for x in exs[1:]:
            s = s + x
        inv = 1.0 / s
        idx_ref[pl.ds(step * _BN, _BN), :] = jnp.concatenate(idxs, axis=1)
        gate_ref[pl.ds(step * _BN, _BN), :] = jnp.concatenate(
            [x * inv for x in exs], axis=1)

    return pl.pallas_call(
        body,
        grid=(rows // _BN,),
        in_specs=[
            pl.BlockSpec((_BN, d), lambda i: (blk0 + i, 0)),
            pl.BlockSpec((e, d), lambda i: (0, 0)),
        ],
        out_specs=[
            pl.BlockSpec((rows, TOP_K), lambda i: (0, 0)),
            pl.BlockSpec((rows, TOP_K), lambda i: (0, 0)),
        ],
        out_shape=[
            jax.ShapeDtypeStruct((rows, TOP_K), jnp.int32),
            jax.ShapeDtypeStruct((rows, TOP_K), jnp.float32),
        ],
    )(hidden_states, w)


_SC_CHUNK_ROWS = (4096, 4096, 4096)
_TC_TAIL_ROWS = 4096


def kernel(hidden_states, W):
    idx_parts, gate_parts = [], []
    row_start = 0
    for rows in _SC_CHUNK_ROWS:
        logits_c = _router_logits(hidden_states, W, row_start, rows)
        idx_c, gates_c = _topk_softmax_sc(logits_c)
        idx_parts.append(idx_c)
        gate_parts.append(gates_c)
        row_start += rows
    idx_t, gates_t = _fused_router_topk_tc(
        hidden_states, W, row_start, _TC_TAIL_ROWS)
    idx_parts.append(idx_t)
    gate_parts.append(gates_t)
    return (jnp.concatenate(idx_parts, axis=0),
            jnp.concatenate(gate_parts, axis=0))


# speedup vs baseline: 1.0444x; 1.0444x over previous
import functools

import jax
import jax.numpy as jnp
from jax import lax
from jax.experimental import pallas as pl
from jax.experimental.pallas import tpu as pltpu
from jax.experimental.pallas import tpu_sc as plsc

TOP_K = 8

_SORT8 = ((0, 1), (2, 3), (4, 5), (6, 7),
          (0, 2), (1, 3), (4, 6), (5, 7),
          (1, 2), (5, 6),
          (0, 4), (1, 5), (2, 6), (3, 7),
          (2, 4), (3, 5),
          (1, 2), (3, 4), (5, 6))
_MERGE8 = ((0, 4), (1, 5), (2, 6), (3, 7),
           (0, 2), (1, 3), (4, 6), (5, 7),
           (0, 1), (2, 3), (4, 5), (6, 7))


def _ce(vals, idxs, i, j):
    m = vals[j] > vals[i]
    vi, vj = vals[i], vals[j]
    ii, ij = idxs[i], idxs[j]
    vals[i] = jnp.where(m, vj, vi)
    vals[j] = jnp.where(m, vi, vj)
    idxs[i] = jnp.where(m, ij, ii)
    idxs[j] = jnp.where(m, ii, ij)

_NC = 2
_NS = 16
_LANES = 16
_NW = _NC * _NS

_BN = 512


def _router_logits(hidden_states, w, row_start, rows):
    t, d = hidden_states.shape
    e = w.shape[0]
    blk0 = row_start // _BN

    def body(h_ref, w_ref, o_ref):
        o_ref[...] = lax.dot_general(
            h_ref[...], w_ref[...],
            (((1,), (1,)), ((), ())),
            preferred_element_type=jnp.float32,
        )

    return pl.pallas_call(
        body,
        grid=(rows // _BN,),
        compiler_params=pltpu.CompilerParams(
            dimension_semantics=("parallel",)),
        in_specs=[
            pl.BlockSpec((_BN, d), lambda i: (blk0 + i, 0)),
            pl.BlockSpec((e, d), lambda i: (0, 0)),
        ],
        out_specs=pl.BlockSpec((_BN, e), lambda i: (i, 0)),
        out_shape=jax.ShapeDtypeStruct((rows, e), jnp.float32),
    )(hidden_states, w)


def _topk_softmax_sc(logits):
    t, e = logits.shape
    rpt = t // _NW
    groups = rpt // _LANES

    mesh = plsc.VectorSubcoreMesh(
        core_axis_name="c", subcore_axis_name="s",
        num_cores=_NC, num_subcores=_NS,
    )

    @functools.partial(
        pl.kernel,
        out_type=(
            jax.ShapeDtypeStruct((t * TOP_K,), jnp.int32),
            jax.ShapeDtypeStruct((t * TOP_K,), jnp.float32),
        ),
        mesh=mesh,
        compiler_params=pltpu.CompilerParams(needs_layout_passes=False),
        scratch_types=[
            pltpu.VMEM((rpt * e,), jnp.float32),
            pltpu.VMEM((rpt * TOP_K,), jnp.int32),
            pltpu.VMEM((rpt * TOP_K,), jnp.float32),
            pltpu.SemaphoreType.DMA,
            pltpu.SemaphoreType.DMA,
        ],
    )
    def run(logits_hbm, idx_hbm, gate_hbm, lg_v, idx_v, gate_v, sem0, sem1):
        wid = lax.axis_index("s") * _NC + lax.axis_index("c")
        base = wid * rpt
        half = (rpt // 2) * e
        cp0 = pltpu.async_copy(
            logits_hbm.at[pl.ds(base * e, half)],
            lg_v.at[pl.ds(0, half)], sem0)
        cp1 = pltpu.async_copy(
            logits_hbm.at[pl.ds(base * e + half, half)],
            lg_v.at[pl.ds(half, half)], sem1)
        cp0.wait()

        def group(g, carry):
            rows = g * _LANES + lax.iota(jnp.int32, _LANES)
            rows_e = rows * e
            rows_k = rows * TOP_K
            tv = ti = None
            for b in range(e // TOP_K):
                bv = [plsc.load_gather(lg_v, [rows_e + (b * TOP_K + k)])
                      for k in range(TOP_K)]
                bi = [jnp.full((_LANES,), b * TOP_K + k, jnp.int32)
                      for k in range(TOP_K)]
                for (i, j) in _SORT8:
                    _ce(bv, bi, i, j)
                if tv is None:
                    tv, ti = bv, bi
                else:
                    cv, ci = [], []
                    for j in range(TOP_K):
                        m = tv[j] > bv[TOP_K - 1 - j]
                        cv.append(jnp.where(m, tv[j], bv[TOP_K - 1 - j]))
                        ci.append(jnp.where(m, ti[j], bi[TOP_K - 1 - j]))
                    for (i, j) in _MERGE8:
                        _ce(cv, ci, i, j)
                    tv, ti = cv, ci
            mx = tv[0]
            ex = [jnp.exp(tj - mx) for tj in tv]
            s = ex[0]
            for j in range(1, TOP_K):
                s = s + ex[j]
            inv = 1.0 / s
            for j in range(TOP_K):
                plsc.store_scatter(idx_v, [rows_k + j], ti[j])
                plsc.store_scatter(gate_v, [rows_k + j], ex[j] * inv)
            return carry

        lax.fori_loop(0, groups // 2, group, 0)
        cp1.wait()
        lax.fori_loop(groups // 2, groups, group, 0)
        pltpu.sync_copy(idx_v, idx_hbm.at[pl.ds(base * TOP_K, rpt * TOP_K)])
        pltpu.sync_copy(gate_v, gate_hbm.at[pl.ds(base * TOP_K, rpt * TOP_K)])

    idx_flat, gate_flat = run(logits.reshape(t * e))
    return idx_flat.reshape(t, TOP_K), gate_flat.reshape(t, TOP_K)


_CHUNK_ROWS = (4096, 4096, 4096, 4096)


def kernel(hidden_states, W):
    idx_parts, gate_parts = [], []
    row_start = 0
    for rows in _CHUNK_ROWS:
        logits_c = _router_logits(hidden_states, W, row_start, rows)
        idx_c, gates_c = _topk_softmax_sc(logits_c)
        idx_parts.append(idx_c)
        gate_parts.append(gates_c)
        row_start += rows
    return (jnp.concatenate(idx_parts, axis=0),
            jnp.concatenate(gate_parts, axis=0))
